# Initial kernel scaffold; baseline (speedup 1.0000x reference)
import functools

import jax
import jax.numpy as jnp
from jax import lax
from jax.experimental import pallas as pl
from jax.experimental.pallas import tpu as pltpu
from jax.experimental.pallas import tpu_sc as plsc

NC = 2
NS = 16
NW = NC * NS
LK_SLOPE = 0.2


def _sc_mesh():
    return plsc.VectorSubcoreMesh(
        core_axis_name="c", subcore_axis_name="s", num_cores=NC, num_subcores=NS
    )


def _sc_degree(row, n_pad, e, k):
    ew = e // NW
    nchunks = ew // k
    rpt = n_pad // NS

    @functools.partial(
        pl.kernel,
        out_type=jax.ShapeDtypeStruct((NC, n_pad), jnp.float32),
        mesh=_sc_mesh(),
        scratch_types=[
            pltpu.VMEM((k,), jnp.int32),
            pltpu.VMEM((k,), jnp.float32),
            pltpu.VMEM((rpt,), jnp.float32),
            pltpu.VMEM_SHARED((n_pad,), jnp.float32),
        ],
    )
    def deg_kernel(row_hbm, out_hbm, idx_v, ones_v, buf_v, acc_s):
        ci = lax.axis_index("c")
        si = lax.axis_index("s")
        wid = si * NC + ci

        def fill_ones(i, _):
            ones_v[pl.ds(i * 16, 16)] = jnp.full((16,), 1.0, jnp.float32)
            return 0

        lax.fori_loop(0, k // 16, fill_ones, 0)

        def fill_zero(i, _):
            buf_v[pl.ds(i * 16, 16)] = jnp.zeros((16,), jnp.float32)
            return 0

        lax.fori_loop(0, rpt // 16, fill_zero, 0)

        pltpu.sync_copy(buf_v, acc_s.at[pl.ds(si * rpt, rpt)])
        plsc.subcore_barrier()

        base = wid * ew

        def body(i, _):
            pltpu.sync_copy(row_hbm.at[pl.ds(base + i * k, k)], idx_v)
            pltpu.sync_copy(ones_v, acc_s.at[idx_v], add=True)
            return 0

        lax.fori_loop(0, nchunks, body, 0)
        plsc.subcore_barrier()

        pltpu.sync_copy(acc_s.at[pl.ds(si * rpt, rpt)], buf_v)
        pltpu.sync_copy(buf_v, out_hbm.at[ci, pl.ds(si * rpt, rpt)])

    return deg_kernel(row)


def _sc_gather_scatter(gtab, row, col, n_pad, d, e, k):
    ew = e // NW
    nchunks = ew // k
    rpt = n_pad // NS

    @functools.partial(
        pl.kernel,
        out_type=jax.ShapeDtypeStruct((NC, n_pad, d), jnp.float32),
        mesh=_sc_mesh(),
        scratch_types=[
            pltpu.VMEM((k,), jnp.int32),
            pltpu.VMEM((k,), jnp.int32),
            pltpu.VMEM((k, d), jnp.float32),
            pltpu.VMEM((rpt, d), jnp.float32),
            pltpu.VMEM_SHARED((n_pad, d), jnp.float32),
            pltpu.SemaphoreType.DMA,
        ],
    )
    def gs_kernel(gtab_hbm, row_hbm, col_hbm, out_hbm,
                  idxr_v, idxc_v, rows_v, buf_v, acc_s, sem):
        ci = lax.axis_index("c")
        si = lax.axis_index("s")
        wid = si * NC + ci

        def fill_zero(i, _):
            for j in range(d // 16):
                buf_v[i, pl.ds(j * 16, 16)] = jnp.zeros((16,), jnp.float32)
            return 0

        lax.fori_loop(0, rpt, fill_zero, 0)
        pltpu.sync_copy(buf_v, acc_s.at[pl.ds(si * rpt, rpt)])
        plsc.subcore_barrier()

        base = wid * ew

        def body(i, _):
            pltpu.sync_copy(row_hbm.at[pl.ds(base + i * k, k)], idxr_v)
            pltpu.sync_copy(col_hbm.at[pl.ds(base + i * k, k)], idxc_v)
            pltpu.async_copy(gtab_hbm.at[idxr_v], rows_v, sem).wait()
            pltpu.sync_copy(rows_v, acc_s.at[idxc_v], add=True)
            return 0

        lax.fori_loop(0, nchunks, body, 0)
        plsc.subcore_barrier()

        pltpu.sync_copy(acc_s.at[pl.ds(si * rpt, rpt)], buf_v)
        pltpu.sync_copy(buf_v, out_hbm.at[ci, pl.ds(si * rpt, rpt)])

    return gs_kernel(gtab, row, col)


def _lk(a):
    return jnp.where(a >= 0, a, LK_SLOPE * a)


def _tc_stage1(deg_t, xpad, w1t, b1r):
    n_pad = xpad.shape[0]
    h = w1t.shape[1]

    def body(deg_ref, x_ref, w_ref, b_ref, g_ref, dis_ref):
        deg = deg_ref[:, 0:1] + deg_ref[:, 1:2]
        dis = lax.rsqrt(deg)
        hh = jnp.dot(x_ref[...], w_ref[...],
                     preferred_element_type=jnp.float32) + b_ref[...]
        g_ref[...] = jnp.concatenate([dis * hh, hh], axis=1)
        dis_ref[...] = dis

    return pl.pallas_call(
        body,
        out_shape=(
            jax.ShapeDtypeStruct((n_pad, 2 * h), jnp.float32),
            jax.ShapeDtypeStruct((n_pad, 1), jnp.float32),
        ),
    )(deg_t, xpad, w1t, b1r)


def _tc_stage2(s1_part, dis, mw1a_r, mw1b, mb1_r, w2t, b2r):
    n_pad, d1 = s1_part.shape[1], s1_part.shape[2]
    h = d1 // 2
    c = w2t.shape[1]

    def body(s_ref, dis_ref, mwa_ref, mwb_ref, mb_ref, w_ref, b_ref, g_ref):
        s = s_ref[0] + s_ref[1]
        u = lax.dot_general(_lk(mwa_ref[...]), mwb_ref[...],
                            (((1,), (1,)), ((), ())),
                            preferred_element_type=jnp.float32)
        dis = dis_ref[...]
        out1 = u * (dis * s[:, :h]) + mb_ref[...] * s[:, h:]
        out1 = jnp.where(out1 > 0, out1, jnp.exp(out1) - 1.0)
        hh = jnp.dot(out1, w_ref[...],
                     preferred_element_type=jnp.float32) + b_ref[...]
        g_ref[...] = jnp.concatenate([dis * hh, hh], axis=1)

    return pl.pallas_call(
        body,
        out_shape=jax.ShapeDtypeStruct((n_pad, 2 * c), jnp.float32),
    )(s1_part, dis, mw1a_r, mw1b, mb1_r, w2t, b2r)


def _tc_stage3(s2_part, dis, mw2a_r, mw2b, mb2_r):
    n_pad, d2 = s2_part.shape[1], s2_part.shape[2]
    c = d2 // 2

    def body(s_ref, dis_ref, mwa_ref, mwb_ref, mb_ref, o_ref):
        s = s_ref[0] + s_ref[1]
        u = lax.dot_general(_lk(mwa_ref[...]), mwb_ref[...],
                            (((1,), (1,)), ((), ())),
                            preferred_element_type=jnp.float32)
        out = u * (dis_ref[...] * s[:, :c]) + mb_ref[...] * s[:, c:]
        m = jnp.max(out, axis=1, keepdims=True)
        z = out - m
        lse = jnp.log(jnp.sum(jnp.exp(z), axis=1, keepdims=True))
        o_ref[...] = z - lse

    return pl.pallas_call(
        body,
        out_shape=jax.ShapeDtypeStruct((n_pad, c), jnp.float32),
    )(s2_part, dis, mw2a_r, mw2b, mb2_r)


@jax.jit
def kernel(x, edge_index, W1, b1, mw1a, mw1b, mb1, W2, b2, mw2a, mw2b, mb2):
    n, f_in = x.shape
    e = edge_index.shape[1]
    h = W1.shape[0]
    c = W2.shape[0]
    n_pad = 10240
    k = 80

    row = edge_index[0]
    col = edge_index[1]

    xpad = jnp.zeros((n_pad, f_in), x.dtype).at[:n].set(x)

    deg_part = _sc_degree(row, n_pad, e, k)
    deg_t = deg_part.T

    g1, dis = _tc_stage1(deg_t, xpad, W1.T, b1.reshape(1, h))
    s1_part = _sc_gather_scatter(g1, row, col, n_pad, 2 * h, e, k)

    g2 = _tc_stage2(s1_part, dis, mw1a.reshape(1, h), mw1b,
                    mb1.reshape(1, h), W2.T, b2.reshape(1, c))
    s2_part = _sc_gather_scatter(g2, row, col, n_pad, 2 * c, e, k)

    out = _tc_stage3(s2_part, dis, mw2a.reshape(1, c), mw2b, mb2.reshape(1, c))
    return out[:n]


# speedup vs baseline: 13.2212x; 13.2212x over previous
import functools

import jax
import jax.numpy as jnp
from jax import lax
from jax.experimental import pallas as pl
from jax.experimental.pallas import tpu as pltpu
from jax.experimental.pallas import tpu_sc as plsc

NC = 2
NS = 16
NW = NC * NS
LK_SLOPE = 0.2


def _sc_mesh():
    return plsc.VectorSubcoreMesh(
        core_axis_name="c", subcore_axis_name="s", num_cores=NC, num_subcores=NS
    )


def _sc_degree(row, n_pad, e, k):
    ew = e // NW
    nchunks = ew // k
    rpt = n_pad // NS

    @functools.partial(
        pl.kernel,
        out_type=jax.ShapeDtypeStruct((NC, n_pad), jnp.float32),
        mesh=_sc_mesh(),
        scratch_types=[
            pltpu.VMEM((k,), jnp.int32),
            pltpu.VMEM((k,), jnp.float32),
            pltpu.VMEM((rpt,), jnp.float32),
            pltpu.VMEM_SHARED((n_pad,), jnp.float32),
        ],
        compiler_params=pltpu.CompilerParams(use_tc_tiling_on_sc=False),
    )
    def deg_kernel(row_hbm, out_hbm, idx_v, ones_v, buf_v, acc_s):
        ci = lax.axis_index("c")
        si = lax.axis_index("s")
        wid = si * NC + ci

        def fill_ones(i, _):
            ones_v[pl.ds(i * 16, 16)] = jnp.full((16,), 1.0, jnp.float32)
            return 0

        lax.fori_loop(0, k // 16, fill_ones, 0)

        def fill_zero(i, _):
            buf_v[pl.ds(i * 16, 16)] = jnp.zeros((16,), jnp.float32)
            return 0

        lax.fori_loop(0, rpt // 16, fill_zero, 0)

        pltpu.sync_copy(buf_v, acc_s.at[pl.ds(si * rpt, rpt)])
        plsc.subcore_barrier()

        base = wid * ew

        def body(i, _):
            pltpu.sync_copy(row_hbm.at[pl.ds(base + i * k, k)], idx_v)
            pltpu.sync_copy(ones_v, acc_s.at[idx_v], add=True)
            return 0

        lax.fori_loop(0, nchunks, body, 0)
        plsc.subcore_barrier()

        pltpu.sync_copy(acc_s.at[pl.ds(si * rpt, rpt)], buf_v)
        pltpu.sync_copy(buf_v, out_hbm.at[ci, pl.ds(si * rpt, rpt)])

    return deg_kernel(row)


def _sc_gather_scatter(gtab, row, col, n_pad, d, e, k):
    ew = e // NW
    nchunks = ew // k
    rpt = n_pad // NS

    @functools.partial(
        pl.kernel,
        out_type=jax.ShapeDtypeStruct((NC, n_pad, d), jnp.float32),
        mesh=_sc_mesh(),
        scratch_types=[
            pltpu.VMEM((k,), jnp.int32),
            pltpu.VMEM((k,), jnp.int32),
            pltpu.VMEM((k, d), jnp.float32),
            pltpu.VMEM((rpt, d), jnp.float32),
            pltpu.VMEM_SHARED((n_pad, d), jnp.float32),
            pltpu.SemaphoreType.DMA,
        ],
        compiler_params=pltpu.CompilerParams(use_tc_tiling_on_sc=False),
    )
    def gs_kernel(gtab_hbm, row_hbm, col_hbm, out_hbm,
                  idxr_v, idxc_v, rows_v, buf_v, acc_s, sem):
        ci = lax.axis_index("c")
        si = lax.axis_index("s")
        wid = si * NC + ci

        def fill_zero(i, _):
            for j in range(d // 16):
                buf_v[i, pl.ds(j * 16, 16)] = jnp.zeros((16,), jnp.float32)
            return 0

        lax.fori_loop(0, rpt, fill_zero, 0)
        pltpu.sync_copy(buf_v, acc_s.at[pl.ds(si * rpt, rpt)])
        plsc.subcore_barrier()

        base = wid * ew

        def body(i, _):
            pltpu.sync_copy(row_hbm.at[pl.ds(base + i * k, k)], idxr_v)
            pltpu.sync_copy(col_hbm.at[pl.ds(base + i * k, k)], idxc_v)
            pltpu.async_copy(gtab_hbm.at[idxr_v], rows_v, sem).wait()
            pltpu.sync_copy(rows_v, acc_s.at[idxc_v], add=True)
            return 0

        lax.fori_loop(0, nchunks, body, 0)
        plsc.subcore_barrier()

        pltpu.sync_copy(acc_s.at[pl.ds(si * rpt, rpt)], buf_v)
        pltpu.sync_copy(buf_v, out_hbm.at[ci, pl.ds(si * rpt, rpt)])

    return gs_kernel(gtab, row, col)


def _lk(a):
    return jnp.where(a >= 0, a, LK_SLOPE * a)


def _tc_stage1(deg_t, xpad, w1t, b1r):
    n_pad = xpad.shape[0]
    h = w1t.shape[1]

    def body(deg_ref, x_ref, w_ref, b_ref, g_ref, dis_ref):
        deg = deg_ref[:, 0:1] + deg_ref[:, 1:2]
        dis = lax.rsqrt(deg)
        hh = jnp.dot(x_ref[...], w_ref[...],
                     preferred_element_type=jnp.float32) + b_ref[...]
        g_ref[...] = jnp.concatenate([dis * hh, hh], axis=1)
        dis_ref[...] = dis

    return pl.pallas_call(
        body,
        out_shape=(
            jax.ShapeDtypeStruct((n_pad, 2 * h), jnp.float32),
            jax.ShapeDtypeStruct((n_pad, 1), jnp.float32),
        ),
    )(deg_t, xpad, w1t, b1r)


def _tc_stage2(s1_part, dis, mw1a_r, mw1b, mb1_r, w2t, b2r):
    n_pad, d1 = s1_part.shape[1], s1_part.shape[2]
    h = d1 // 2
    c = w2t.shape[1]

    def body(s_ref, dis_ref, mwa_ref, mwb_ref, mb_ref, w_ref, b_ref, g_ref):
        s = s_ref[0] + s_ref[1]
        u = lax.dot_general(_lk(mwa_ref[...]), mwb_ref[...],
                            (((1,), (1,)), ((), ())),
                            preferred_element_type=jnp.float32)
        dis = dis_ref[...]
        out1 = u * (dis * s[:, :h]) + mb_ref[...] * s[:, h:]
        out1 = jnp.where(out1 > 0, out1, jnp.exp(out1) - 1.0)
        hh = jnp.dot(out1, w_ref[...],
                     preferred_element_type=jnp.float32) + b_ref[...]
        g_ref[...] = jnp.concatenate([dis * hh, hh], axis=1)

    return pl.pallas_call(
        body,
        out_shape=jax.ShapeDtypeStruct((n_pad, 2 * c), jnp.float32),
    )(s1_part, dis, mw1a_r, mw1b, mb1_r, w2t, b2r)


def _tc_stage3(s2_part, dis, mw2a_r, mw2b, mb2_r):
    n_pad, d2 = s2_part.shape[1], s2_part.shape[2]
    c = d2 // 2

    def body(s_ref, dis_ref, mwa_ref, mwb_ref, mb_ref, o_ref):
        s = s_ref[0] + s_ref[1]
        u = lax.dot_general(_lk(mwa_ref[...]), mwb_ref[...],
                            (((1,), (1,)), ((), ())),
                            preferred_element_type=jnp.float32)
        out = u * (dis_ref[...] * s[:, :c]) + mb_ref[...] * s[:, c:]
        m = jnp.max(out, axis=1, keepdims=True)
        z = out - m
        lse = jnp.log(jnp.sum(jnp.exp(z), axis=1, keepdims=True))
        o_ref[...] = z - lse

    return pl.pallas_call(
        body,
        out_shape=jax.ShapeDtypeStruct((n_pad, c), jnp.float32),
    )(s2_part, dis, mw2a_r, mw2b, mb2_r)


@jax.jit
def kernel(x, edge_index, W1, b1, mw1a, mw1b, mb1, W2, b2, mw2a, mw2b, mb2):
    n, f_in = x.shape
    e = edge_index.shape[1]
    h = W1.shape[0]
    c = W2.shape[0]
    n_pad = 10240
    k = 80

    row = edge_index[0]
    col = edge_index[1]

    xpad = jnp.zeros((n_pad, f_in), x.dtype).at[:n].set(x)

    deg_part = _sc_degree(row, n_pad, e, k)
    deg_t = deg_part.T

    g1, dis = _tc_stage1(deg_t, xpad, W1.T, b1.reshape(1, h))
    s1_part = _sc_gather_scatter(g1, row, col, n_pad, 2 * h, e, k)

    g2 = _tc_stage2(s1_part, dis, mw1a.reshape(1, h), mw1b,
                    mb1.reshape(1, h), W2.T, b2.reshape(1, c))
    s2_part = _sc_gather_scatter(g2, row, col, n_pad, 2 * c, e, k)

    out = _tc_stage3(s2_part, dis, mw2a.reshape(1, c), mw2b, mb2.reshape(1, c))
    return out[:n]


# speedup vs baseline: 15.3027x; 1.1574x over previous
import functools

import jax
import jax.numpy as jnp
from jax import lax
from jax.experimental import pallas as pl
from jax.experimental.pallas import tpu as pltpu
from jax.experimental.pallas import tpu_sc as plsc

NC = 2
NS = 16
NW = NC * NS
LK_SLOPE = 0.2


def _sc_mesh():
    return plsc.VectorSubcoreMesh(
        core_axis_name="c", subcore_axis_name="s", num_cores=NC, num_subcores=NS
    )


def _sc_degree(row3, n_pad, nch, k):
    rpt = n_pad // NS

    @functools.partial(
        pl.kernel,
        out_type=jax.ShapeDtypeStruct((NC, n_pad), jnp.float32),
        mesh=_sc_mesh(),
        scratch_types=[
            pltpu.VMEM((nch, k), jnp.int32),
            pltpu.VMEM((k,), jnp.float32),
            pltpu.VMEM((rpt,), jnp.float32),
            pltpu.VMEM_SHARED((n_pad,), jnp.float32),
        ],
        compiler_params=pltpu.CompilerParams(use_tc_tiling_on_sc=False),
    )
    def deg_kernel(row_hbm, out_hbm, idx_v, ones_v, buf_v, acc_s):
        ci = lax.axis_index("c")
        si = lax.axis_index("s")
        wid = si * NC + ci

        def fill_ones(i, _):
            ones_v[pl.ds(i * 16, 16)] = jnp.full((16,), 1.0, jnp.float32)
            return 0

        lax.fori_loop(0, k // 16, fill_ones, 0)

        def fill_zero(i, _):
            buf_v[pl.ds(i * 16, 16)] = jnp.zeros((16,), jnp.float32)
            return 0

        lax.fori_loop(0, rpt // 16, fill_zero, 0)

        pltpu.sync_copy(row_hbm.at[wid], idx_v)
        pltpu.sync_copy(buf_v, acc_s.at[pl.ds(si * rpt, rpt)])
        plsc.subcore_barrier()

        def body(i, _):
            pltpu.sync_copy(ones_v, acc_s.at[idx_v.at[i]], add=True)
            return 0

        lax.fori_loop(0, nch, body, 0)
        plsc.subcore_barrier()

        pltpu.sync_copy(acc_s.at[pl.ds(si * rpt, rpt)], buf_v)
        pltpu.sync_copy(buf_v, out_hbm.at[ci, pl.ds(si * rpt, rpt)])

    return deg_kernel(row3)


def _sc_gather_scatter(gtab, row3, col3, n_pad, d, nch, k):
    rpt = n_pad // NS
    NB = 4
    assert nch % NB == 0

    @functools.partial(
        pl.kernel,
        out_type=jax.ShapeDtypeStruct((NC, n_pad, d), jnp.float32),
        mesh=_sc_mesh(),
        scratch_types=[
            pltpu.VMEM((nch, k), jnp.int32),
            pltpu.VMEM((nch, k), jnp.int32),
            [pltpu.VMEM((k, d), jnp.float32)] * NB,
            pltpu.VMEM((rpt, d), jnp.float32),
            pltpu.VMEM_SHARED((n_pad, d), jnp.float32),
            [pltpu.SemaphoreType.DMA] * NB,
        ],
        compiler_params=pltpu.CompilerParams(use_tc_tiling_on_sc=False),
    )
    def gs_kernel(gtab_hbm, row_hbm, col_hbm, out_hbm,
                  row_v, col_v, bufs, buf_v, acc_s, sems):
        ci = lax.axis_index("c")
        si = lax.axis_index("s")
        wid = si * NC + ci

        def fill_zero(i, _):
            for j in range(d // 16):
                buf_v[i, pl.ds(j * 16, 16)] = jnp.zeros((16,), jnp.float32)
            return 0

        lax.fori_loop(0, rpt, fill_zero, 0)
        pltpu.sync_copy(row_hbm.at[wid], row_v)
        pltpu.sync_copy(col_hbm.at[wid], col_v)
        pltpu.sync_copy(buf_v, acc_s.at[pl.ds(si * rpt, rpt)])
        plsc.subcore_barrier()

        for p in range(NB - 1):
            pltpu.async_copy(gtab_hbm.at[row_v.at[p]], bufs[p], sems[p])

        def body(j, _):
            for p in range(NB):
                i = j * NB + p
                pltpu.make_async_copy(
                    gtab_hbm.at[row_v.at[i]], bufs[p], sems[p]).wait()
                nxt = i + NB - 1
                q = (p + NB - 1) % NB

                @pl.when(nxt < nch)
                def _prefetch():
                    pltpu.async_copy(
                        gtab_hbm.at[row_v.at[nxt]], bufs[q], sems[q])

                pltpu.sync_copy(bufs[p], acc_s.at[col_v.at[i]], add=True)
            return 0

        lax.fori_loop(0, nch // NB, body, 0)
        plsc.subcore_barrier()

        pltpu.sync_copy(acc_s.at[pl.ds(si * rpt, rpt)], buf_v)
        pltpu.sync_copy(buf_v, out_hbm.at[ci, pl.ds(si * rpt, rpt)])

    return gs_kernel(gtab, row3, col3)


def _lk(a):
    return jnp.where(a >= 0, a, LK_SLOPE * a)


def _tc_stage1(deg_t, xpad, w1t, b1r):
    n_pad = xpad.shape[0]
    h = w1t.shape[1]

    def body(deg_ref, x_ref, w_ref, b_ref, g_ref, dis_ref):
        deg = deg_ref[:, 0:1] + deg_ref[:, 1:2]
        dis = lax.rsqrt(deg)
        hh = jnp.dot(x_ref[...], w_ref[...],
                     preferred_element_type=jnp.float32) + b_ref[...]
        g_ref[...] = jnp.concatenate([dis * hh, hh], axis=1)
        dis_ref[...] = dis

    return pl.pallas_call(
        body,
        out_shape=(
            jax.ShapeDtypeStruct((n_pad, 2 * h), jnp.float32),
            jax.ShapeDtypeStruct((n_pad, 1), jnp.float32),
        ),
    )(deg_t, xpad, w1t, b1r)


def _tc_stage2(s1_part, dis, mw1a_r, mw1b, mb1_r, w2t, b2r):
    n_pad, d1 = s1_part.shape[1], s1_part.shape[2]
    h = d1 // 2
    c = w2t.shape[1]

    def body(s_ref, dis_ref, mwa_ref, mwb_ref, mb_ref, w_ref, b_ref, g_ref):
        s = s_ref[0] + s_ref[1]
        u = lax.dot_general(_lk(mwa_ref[...]), mwb_ref[...],
                            (((1,), (1,)), ((), ())),
                            preferred_element_type=jnp.float32)
        dis = dis_ref[...]
        out1 = u * (dis * s[:, :h]) + mb_ref[...] * s[:, h:]
        out1 = jnp.where(out1 > 0, out1, jnp.exp(out1) - 1.0)
        hh = jnp.dot(out1, w_ref[...],
                     preferred_element_type=jnp.float32) + b_ref[...]
        g_ref[...] = jnp.concatenate([dis * hh, hh], axis=1)

    return pl.pallas_call(
        body,
        out_shape=jax.ShapeDtypeStruct((n_pad, 2 * c), jnp.float32),
    )(s1_part, dis, mw1a_r, mw1b, mb1_r, w2t, b2r)


def _tc_stage3(s2_part, dis, mw2a_r, mw2b, mb2_r):
    n_pad, d2 = s2_part.shape[1], s2_part.shape[2]
    c = d2 // 2

    def body(s_ref, dis_ref, mwa_ref, mwb_ref, mb_ref, o_ref):
        s = s_ref[0] + s_ref[1]
        u = lax.dot_general(_lk(mwa_ref[...]), mwb_ref[...],
                            (((1,), (1,)), ((), ())),
                            preferred_element_type=jnp.float32)
        out = u * (dis_ref[...] * s[:, :c]) + mb_ref[...] * s[:, c:]
        m = jnp.max(out, axis=1, keepdims=True)
        z = out - m
        lse = jnp.log(jnp.sum(jnp.exp(z), axis=1, keepdims=True))
        o_ref[...] = z - lse

    return pl.pallas_call(
        body,
        out_shape=jax.ShapeDtypeStruct((n_pad, c), jnp.float32),
    )(s2_part, dis, mw2a_r, mw2b, mb2_r)


@jax.jit
def kernel(x, edge_index, W1, b1, mw1a, mw1b, mb1, W2, b2, mw2a, mw2b, mb2):
    n, f_in = x.shape
    e = edge_index.shape[1]
    h = W1.shape[0]
    c = W2.shape[0]
    n_pad = 10240
    k = 128
    nch = -(-e // (NW * k) - 1) // 4 * 4 + 4
    e_pad = NW * nch * k

    pad = jnp.full((2, e_pad - e), n_pad - 1, jnp.int32)
    ei = jnp.concatenate([edge_index, pad], axis=1)
    row3 = ei[0].reshape(NW, nch, k)
    col3 = ei[1].reshape(NW, nch, k)

    xpad = jnp.zeros((n_pad, f_in), x.dtype).at[:n].set(x)

    deg_part = _sc_degree(row3, n_pad, nch, k)
    deg_t = deg_part.T

    g1, dis = _tc_stage1(deg_t, xpad, W1.T, b1.reshape(1, h))
    s1_part = _sc_gather_scatter(g1, row3, col3, n_pad, 2 * h, nch, k)

    g2 = _tc_stage2(s1_part, dis, mw1a.reshape(1, h), mw1b,
                    mb1.reshape(1, h), W2.T, b2.reshape(1, c))
    s2_part = _sc_gather_scatter(g2, row3, col3, n_pad, 2 * c, nch, k)

    out = _tc_stage3(s2_part, dis, mw2a.reshape(1, c), mw2b, mb2.reshape(1, c))
    return out[:n]


# speedup vs baseline: 17.6322x; 1.1522x over previous
import functools

import jax
import jax.numpy as jnp
from jax import lax
from jax.experimental import pallas as pl
from jax.experimental.pallas import tpu as pltpu
from jax.experimental.pallas import tpu_sc as plsc

NC = 2
NS = 16
NW = NC * NS
LK_SLOPE = 0.2


def _sc_mesh():
    return plsc.VectorSubcoreMesh(
        core_axis_name="c", subcore_axis_name="s", num_cores=NC, num_subcores=NS
    )


def _sc_degree(row3, n_pad, nch, k):
    rpt = n_pad // NS

    @functools.partial(
        pl.kernel,
        out_type=jax.ShapeDtypeStruct((NC, n_pad), jnp.float32),
        mesh=_sc_mesh(),
        scratch_types=[
            pltpu.VMEM((nch, k), jnp.int32),
            pltpu.VMEM((k,), jnp.float32),
            pltpu.VMEM((rpt,), jnp.float32),
            pltpu.VMEM_SHARED((n_pad,), jnp.float32),
        ],
        compiler_params=pltpu.CompilerParams(use_tc_tiling_on_sc=False),
    )
    def deg_kernel(row_hbm, out_hbm, idx_v, ones_v, buf_v, acc_s):
        ci = lax.axis_index("c")
        si = lax.axis_index("s")
        wid = si * NC + ci

        def fill_ones(i, _):
            ones_v[pl.ds(i * 16, 16)] = jnp.full((16,), 1.0, jnp.float32)
            return 0

        lax.fori_loop(0, k // 16, fill_ones, 0)

        def fill_zero(i, _):
            buf_v[pl.ds(i * 16, 16)] = jnp.zeros((16,), jnp.float32)
            return 0

        lax.fori_loop(0, rpt // 16, fill_zero, 0)

        pltpu.sync_copy(row_hbm.at[wid], idx_v)
        pltpu.sync_copy(buf_v, acc_s.at[pl.ds(si * rpt, rpt)])
        plsc.subcore_barrier()

        def body(i, _):
            pltpu.sync_copy(ones_v, acc_s.at[idx_v.at[i]], add=True)
            return 0

        lax.fori_loop(0, nch, body, 0)
        plsc.subcore_barrier()

        pltpu.sync_copy(acc_s.at[pl.ds(si * rpt, rpt)], buf_v)
        pltpu.sync_copy(buf_v, out_hbm.at[ci, pl.ds(si * rpt, rpt)])

    return deg_kernel(row3)


def _sc_gather_scatter(gtab, row3, col3, n_pad, d, nch, k):
    rpt = n_pad // NS
    NB = 4
    assert nch % NB == 0

    @functools.partial(
        pl.kernel,
        out_type=jax.ShapeDtypeStruct((NC, n_pad, d), jnp.float32),
        mesh=_sc_mesh(),
        scratch_types=[
            pltpu.VMEM((nch, k), jnp.int32),
            pltpu.VMEM((nch, k), jnp.int32),
            [pltpu.VMEM((k, d), jnp.float32)] * NB,
            pltpu.VMEM((rpt, d), jnp.float32),
            pltpu.VMEM_SHARED((n_pad, d), jnp.float32),
            [pltpu.SemaphoreType.DMA] * NB,
        ],
        compiler_params=pltpu.CompilerParams(use_tc_tiling_on_sc=False),
    )
    def gs_kernel(gtab_hbm, row_hbm, col_hbm, out_hbm,
                  row_v, col_v, bufs, buf_v, acc_s, sems):
        ci = lax.axis_index("c")
        si = lax.axis_index("s")
        wid = si * NC + ci

        zoffs = sorted({min(j * 16, d - 16) for j in range(-(-d // 16))})

        def fill_zero(i, _):
            for off in zoffs:
                buf_v[i, pl.ds(off, 16)] = jnp.zeros((16,), jnp.float32)
            return 0

        lax.fori_loop(0, rpt, fill_zero, 0)
        pltpu.sync_copy(row_hbm.at[wid], row_v)
        pltpu.sync_copy(col_hbm.at[wid], col_v)
        pltpu.sync_copy(buf_v, acc_s.at[pl.ds(si * rpt, rpt)])
        plsc.subcore_barrier()

        for p in range(NB - 1):
            pltpu.async_copy(gtab_hbm.at[row_v.at[p]], bufs[p], sems[p])

        def body(j, _):
            for p in range(NB):
                i = j * NB + p
                pltpu.make_async_copy(
                    gtab_hbm.at[row_v.at[i]], bufs[p], sems[p]).wait()
                nxt = i + NB - 1
                q = (p + NB - 1) % NB

                @pl.when(nxt < nch)
                def _prefetch():
                    pltpu.async_copy(
                        gtab_hbm.at[row_v.at[nxt]], bufs[q], sems[q])

                pltpu.sync_copy(bufs[p], acc_s.at[col_v.at[i]], add=True)
            return 0

        lax.fori_loop(0, nch // NB, body, 0)
        plsc.subcore_barrier()

        pltpu.sync_copy(acc_s.at[pl.ds(si * rpt, rpt)], buf_v)
        pltpu.sync_copy(buf_v, out_hbm.at[ci, pl.ds(si * rpt, rpt)])

    return gs_kernel(gtab, row3, col3)


def _lk(a):
    return jnp.where(a >= 0, a, LK_SLOPE * a)


def _tc_stage1(deg_t, xpad, w1t, b1r):
    n_pad = xpad.shape[0]
    h = w1t.shape[1]

    def body(deg_ref, x_ref, w_ref, b_ref, g_ref, dis_ref):
        deg = deg_ref[:, 0:1] + deg_ref[:, 1:2]
        dis = lax.rsqrt(deg)
        hh = jnp.dot(x_ref[...], w_ref[...],
                     preferred_element_type=jnp.float32) + b_ref[...]
        one = jnp.ones_like(dis)
        g_ref[...] = jnp.concatenate([dis * hh, hh, dis, one], axis=1)
        dis_ref[...] = dis

    return pl.pallas_call(
        body,
        out_shape=(
            jax.ShapeDtypeStruct((n_pad, 2 * h + 2), jnp.float32),
            jax.ShapeDtypeStruct((n_pad, 1), jnp.float32),
        ),
    )(deg_t, xpad, w1t, b1r)


def _tc_stage2(s1_part, dis, mw1a_r, mw1b, mb1_r):
    n_pad, d1 = s1_part.shape[1], s1_part.shape[2]
    h = (d1 - 2) // 2

    def body(s_ref, dis_ref, mwa_ref, mwb_ref, mb_ref, g_ref):
        s = s_ref[0] + s_ref[1]
        u = lax.dot_general(_lk(mwa_ref[...]), mwb_ref[...],
                            (((1,), (1,)), ((), ())),
                            preferred_element_type=jnp.float32)
        dis = dis_ref[...]
        out1 = u * (dis * s[:, :h]) + mb_ref[...] * s[:, h:2 * h]
        out1 = jnp.where(out1 > 0, out1, jnp.exp(out1) - 1.0)
        g_ref[...] = jnp.concatenate([dis * out1, out1], axis=1)

    return pl.pallas_call(
        body,
        out_shape=jax.ShapeDtypeStruct((n_pad, 2 * h), jnp.float32),
    )(s1_part, dis, mw1a_r, mw1b, mb1_r)


def _tc_stage3(s2_part, s1_part, dis, w2t, b2r, mw2a_r, mw2b, mb2_r):
    n_pad, d2 = s2_part.shape[1], s2_part.shape[2]
    h = d2 // 2
    c = w2t.shape[1]
    d1 = s1_part.shape[2]

    def body(s2_ref, s1_ref, dis_ref, w_ref, b_ref,
             mwa_ref, mwb_ref, mb_ref, o_ref):
        s2 = s2_ref[0] + s2_ref[1]
        s1 = s1_ref[0] + s1_ref[1]
        dis = dis_ref[...]
        nsum = dis * s1[:, d1 - 2:d1 - 1]
        degin = s1[:, d1 - 1:d1]
        u = lax.dot_general(_lk(mwa_ref[...]), mwb_ref[...],
                            (((1,), (1,)), ((), ())),
                            preferred_element_type=jnp.float32)
        sa = dis * jnp.dot(s2[:, :h], w_ref[...],
                           preferred_element_type=jnp.float32) + nsum * b_ref[...]
        sb = jnp.dot(s2[:, h:], w_ref[...],
                     preferred_element_type=jnp.float32) + degin * b_ref[...]
        out = u * sa + mb_ref[...] * sb
        m = jnp.max(out, axis=1, keepdims=True)
        z = out - m
        lse = jnp.log(jnp.sum(jnp.exp(z), axis=1, keepdims=True))
        o_ref[...] = z - lse

    return pl.pallas_call(
        body,
        out_shape=jax.ShapeDtypeStruct((n_pad, c), jnp.float32),
    )(s2_part, s1_part, dis, w2t, b2r, mw2a_r, mw2b, mb2_r)


@jax.jit
def kernel(x, edge_index, W1, b1, mw1a, mw1b, mb1, W2, b2, mw2a, mw2b, mb2):
    n, f_in = x.shape
    e = edge_index.shape[1]
    h = W1.shape[0]
    c = W2.shape[0]
    n_pad = 10240
    k = 128
    nch = -(-e // (NW * k) - 1) // 4 * 4 + 4
    e_pad = NW * nch * k

    pad = jnp.full((2, e_pad - e), n_pad - 1, jnp.int32)
    ei = jnp.concatenate([edge_index, pad], axis=1)
    row3 = ei[0].reshape(NW, nch, k)
    col3 = ei[1].reshape(NW, nch, k)

    xpad = jnp.zeros((n_pad, f_in), x.dtype).at[:n].set(x)

    deg_part = _sc_degree(row3, n_pad, nch, k)
    deg_t = deg_part.T

    g1, dis = _tc_stage1(deg_t, xpad, W1.T, b1.reshape(1, h))
    s1_part = _sc_gather_scatter(g1, row3, col3, n_pad, 2 * h + 2, nch, k)

    g2 = _tc_stage2(s1_part, dis, mw1a.reshape(1, h), mw1b, mb1.reshape(1, h))
    s2_part = _sc_gather_scatter(g2, row3, col3, n_pad, 2 * h, nch, k)

    out = _tc_stage3(s2_part, s1_part, dis, W2.T, b2.reshape(1, c),
                     mw2a.reshape(1, c), mw2b, mb2.reshape(1, c))
    return out[:n]
